# baseline (speedup 1.0000x reference)
import numpy as np

import jax
import jax.numpy as jnp
from jax import lax
from jax.experimental import pallas as pl
from jax.experimental.pallas import tpu as pltpu
from jax.experimental.pallas import tpu_sc as plsc

N = 16384
M = 5000
M_PAD = 5120
NCH = M_PAD // 16
KST = 32
CPS = NCH // KST
K = 5
NC = 2
NS = 16
NW = NC * NS
RPW = N // NW
GPW = RPW // 16
R_OUT = 9
R_IN = 5
MASK_SENTINEL = 1e8

Q_STRIPE = 64
NSTR = N // Q_STRIPE
LV = Q_STRIPE // 16
_GIDX = np.empty((NW, GPW), np.int32)
for _s in range(NSTR):
    for _j in range(LV):
        _GIDX[(_s + 8 * _j) % NW, (_s // NW) * LV + _j] = _s * LV + _j
_GIDX_FLAT = _GIDX.reshape(-1)

_GDN = lax.GatherDimensionNumbers(
    offset_dims=(), collapsed_slice_dims=(0,), start_index_map=(0,))


def _shuffle(v, idx):
    return lax.gather(v, idx[:, None], _GDN, slice_sizes=(1,),
                      mode=lax.GatherScatterMode.PROMISE_IN_BOUNDS)


def _knn_body(keys_hbm, rows_hbm, sxb_hbm, out_hbm, keys_v, rows_v, sxb_v,
              out_v):
    wid = lax.axis_index("c") * NS + lax.axis_index("s")
    pltpu.sync_copy(keys_hbm, keys_v)
    pltpu.sync_copy(rows_hbm.at[wid], rows_v)
    pltpu.sync_copy(sxb_hbm, sxb_v)

    ii = lax.iota(jnp.int32, 16)
    iz = ii * 0
    i15 = iz + 15
    one = jnp.int32(1)
    zero = jnp.int32(0)

    def any_lanes(fl, fr):
        v = fl * 32 + fr
        for s in (8, 4, 2, 1):
            v = v + _shuffle(v, jnp.bitwise_xor(ii, s))
        tot = v[0]
        return jnp.minimum(tot // 32, 1), jnp.minimum(tot % 32, 1)

    def sxb(row2, s):
        v = sxb_v[row2 + s // 16, :]
        return _shuffle(v, iz + s % 16)

    def group_body(slot, _):
        qx = rows_v[slot, 0, :]
        qy = rows_v[slot, 1, :]
        qmx = qx[8]
        qmy = qy[8]

        def chunk_chain(c, ms):
            kxv = keys_v[0, pl.ds(c * 16, 16)]
            kyv = keys_v[1, pl.ds(c * 16, 16)]
            m0, m1, m2, m3, m4 = ms
            for i in range(16):
                dx = qx - kxv[i]
                dy = qy - kyv[i]
                d2 = dx * dx + dy * dy
                v = jnp.where(d2 < 1e-4, MASK_SENTINEL, d2)
                n0 = jnp.minimum(m0, v)
                v = jnp.maximum(m0, v)
                n1 = jnp.minimum(m1, v)
                v = jnp.maximum(m1, v)
                n2 = jnp.minimum(m2, v)
                v = jnp.maximum(m2, v)
                n3 = jnp.minimum(m3, v)
                v = jnp.maximum(m3, v)
                m4 = jnp.minimum(m4, v)
                m0, m1, m2, m3 = n0, n1, n2, n3
            return (m0, m1, m2, m3, m4)

        def visit_stripe(s, ms):
            xg = jnp.maximum(jnp.maximum(sxb(0, s) - qx, qx - sxb(2, s)),
                             0.0)
            xg2 = xg * xg
            base = s * CPS
            lo = zero
            hi = jnp.int32(CPS - 1)
            for _u in range(4):
                mid = (lo + hi + one) // 2
                kch = keys_v[1, pl.ds((base + mid) * 16, 16)]
                fge = jnp.where(kch <= qmy, one, zero)[0]
                lo = lo + fge * (mid - lo)
                hi = mid - one + fge * (hi - mid + one)
            c0 = lo
            ms = chunk_chain(base + c0, ms)

            def in_round(r, st):
                cl, cr, m0, m1, m2, m3, m4 = st
                cap = jnp.int32(2) << jnp.minimum(r // 2, 2)
                crc = jnp.minimum(cr, jnp.int32(CPS - 1))
                ylo = _shuffle(keys_v[1, pl.ds((base + crc) * 16, 16)], iz)
                dyr = jnp.maximum(ylo - qy, 0.0)
                validr = jnp.where(cr <= CPS - 1, one, zero)
                fr = jnp.where(xg2 + dyr * dyr < m4, validr, zero)
                clc = jnp.maximum(cl, zero)
                yhi = _shuffle(keys_v[1, pl.ds((base + clc) * 16, 16)], i15)
                dyl = jnp.maximum(qy - yhi, 0.0)
                validl = jnp.where(cl >= 0, one, zero)
                fl = jnp.where(xg2 + dyl * dyl < m4, validl, zero)
                needl, needr = any_lanes(fl, fr)
                nr = jnp.minimum(needr * cap, jnp.int32(CPS) - cr)
                nl = jnp.minimum(needl * cap, cl + one)
                mm = (m0, m1, m2, m3, m4)
                mm = lax.fori_loop(
                    0, nr, lambda i, m: chunk_chain(base + cr + i, m), mm)
                mm = lax.fori_loop(
                    0, nl, lambda i, m: chunk_chain(base + cl - i, m), mm)
                return (cl - nl, cr + nr) + mm

            st = lax.fori_loop(0, R_IN, in_round, (c0 - one, c0 + one) + ms)
            return st[2:]

        f = (jnp.where(sxb_v[0, :] <= qmx, one, zero)
             + jnp.where(sxb_v[1, :] <= qmx, one, zero))
        for sh in (8, 4, 2, 1):
            f = f + _shuffle(f, jnp.bitwise_xor(ii, sh))
        s0 = jnp.clip(f[0] - one, 0, KST - 1)

        inf = jnp.full((16,), jnp.inf, jnp.float32)
        ms = visit_stripe(s0, (inf, inf, inf, inf, inf))

        def out_round(r, st):
            sl, sr, m0, m1, m2, m3, m4 = st
            cap = jnp.int32(1) << jnp.minimum(r // 2, 4)
            src = jnp.minimum(sr, jnp.int32(KST - 1))
            dxr = jnp.maximum(sxb(0, src) - qx, 0.0)
            validr = jnp.where(sr <= KST - 1, one, zero)
            fr = jnp.where(dxr * dxr < m4, validr, zero)
            slc = jnp.maximum(sl, zero)
            dxl = jnp.maximum(qx - sxb(2, slc), 0.0)
            validl = jnp.where(sl >= 0, one, zero)
            fl = jnp.where(dxl * dxl < m4, validl, zero)
            needl, needr = any_lanes(fl, fr)
            nr = jnp.minimum(needr * cap, jnp.int32(KST) - sr)
            nl = jnp.minimum(needl * cap, sl + one)
            mm = (m0, m1, m2, m3, m4)
            mm = lax.fori_loop(
                0, nr, lambda i, m: visit_stripe(sr + i, m), mm)
            mm = lax.fori_loop(
                0, nl, lambda i, m: visit_stripe(sl - i, m), mm)
            return (sl - nl, sr + nr) + mm

        st = lax.fori_loop(0, R_OUT, out_round, (s0 - one, s0 + one) + ms)
        ms = st[2:]
        for t in range(K):
            out_v[slot, t, :] = ms[t]
        return 0

    lax.fori_loop(0, GPW, group_body, 0)
    pltpu.sync_copy(out_v, out_hbm.at[wid])


def _knn_smallest5(keys2xm, rows_arr, sxb):
    mesh = plsc.VectorSubcoreMesh(core_axis_name="c", subcore_axis_name="s",
                                  num_cores=NC, num_subcores=NS)
    fn = pl.kernel(
        _knn_body,
        out_type=jax.ShapeDtypeStruct((NW, GPW, K, 16), jnp.float32),
        mesh=mesh,
        scratch_types=[
            pltpu.VMEM((2, M_PAD), jnp.float32),
            pltpu.VMEM((GPW, 2, 16), jnp.float32),
            pltpu.VMEM((4, 16), jnp.float32),
            pltpu.VMEM((GPW, K, 16), jnp.float32),
        ],
    )
    return fn(keys2xm, rows_arr, sxb)


def _loss_body(c_t, k5_t, sig_s_ref, lc_ref, lkl_ref, lvar_ref):
    mu_t = c_t[0:4, :]
    ls_t = c_t[4:8, :]
    sigma_s = sig_s_ref[0]
    s = c_t[12:13, :]

    gtd = (c_t[10:12, :] - c_t[8:10, :]) / s
    d = mu_t[0:2, :] - gtd
    a = jnp.abs(d)
    sl1 = jnp.where(a < 1.0, 0.5 * d * d, a - 0.5)
    lc_ref[0] = jnp.sum(sl1) / (2.0 * N)

    k5 = k5_t[...]
    d5 = jnp.where(k5 > 5e7, MASK_SENTINEL, jnp.sqrt(k5))
    d_px = jnp.sum(d5, axis=0, keepdims=True) * (1.0 / K)
    d_norm = jnp.clip(d_px / s, 0.5, 16.0)
    log_d = jnp.log(d_norm)
    sigma_c = jnp.maximum(0.5 * d_norm, 1e-6)
    sig_s_eff = jnp.maximum(sigma_s, 1e-6)

    mu01 = mu_t[0:2, :]
    mu23 = mu_t[2:4, :]
    sq01 = jnp.maximum(jnp.exp(ls_t[0:2, :]), 1e-6)
    sq23 = jnp.maximum(jnp.exp(ls_t[2:4, :]), 1e-6)
    kl01 = (jnp.log(sigma_c / sq01)
            + (sq01 * sq01 + mu01 * mu01) / (2.0 * sigma_c * sigma_c) - 0.5)
    dm23 = mu23 - log_d
    kl23 = (jnp.log(sig_s_eff / sq23)
            + (sq23 * sq23 + dm23 * dm23) / (2.0 * sig_s_eff * sig_s_eff)
            - 0.5)
    kl_sample = jnp.minimum(jnp.sum(kl01, axis=0) + jnp.sum(kl23, axis=0),
                            50.0)
    lkl_ref[0] = jnp.sum(kl_sample) / N
    lvar_ref[0] = jnp.sum(jnp.exp(ls_t[0:2, :])) / (2.0 * N)


def _losses(c_t, k5_t, sigma_s):
    return pl.pallas_call(
        _loss_body,
        out_shape=[jax.ShapeDtypeStruct((1,), jnp.float32)] * 3,
        in_specs=[pl.BlockSpec(memory_space=pltpu.VMEM)] * 2
        + [pl.BlockSpec(memory_space=pltpu.SMEM)],
        out_specs=[pl.BlockSpec(memory_space=pltpu.SMEM)] * 3,
    )(c_t, k5_t, sigma_s)


def kernel(bbox_mu, bbox_log_sigma, pos_points, pos_strides, gt_centers,
           gt_centers_list, cur_iter):
    lambda_center = 1.0
    lambda_kl = 0.1
    lambda_kl_warmup = 0.02
    lambda_var = 0.01
    sigma_s_init = 1.0
    sigma_s_final = 0.4
    warmup_iters = 2000
    anneal_iters = 2000

    cur_iter_f = jnp.asarray(cur_iter, jnp.float32)
    ratio = jnp.clip((cur_iter_f - warmup_iters) / anneal_iters, 0.0, 1.0)
    eff_lambda_kl = lambda_kl_warmup + ratio * (lambda_kl - lambda_kl_warmup)
    sigma_s = sigma_s_init - ratio * (sigma_s_init - sigma_s_final)

    all_gt = jnp.reshape(gt_centers_list, (M, 2))
    keys_xs = jnp.pad(all_gt[jnp.argsort(all_gt[:, 0])],
                      ((0, M_PAD - M), (0, 0)),
                      constant_values=1e6)
    kstr = keys_xs.reshape(KST, CPS * 16, 2)
    sxb = jnp.stack([kstr[:, 0, 0].reshape(2, 16),
                     kstr[:, -1, 0].reshape(2, 16)]
                    ).reshape(4, 16)
    yo = jnp.argsort(kstr[:, :, 1], axis=1)
    keys2xm = (jnp.take_along_axis(kstr, yo[:, :, None], axis=1)
               .reshape(M_PAD, 2).T)

    o1 = jnp.argsort(gt_centers[:, 0]).reshape(NSTR, Q_STRIPE)
    o2 = jnp.argsort(gt_centers[:, 1][o1], axis=1)
    order = jnp.take_along_axis(o1, o2, axis=1).reshape(-1)
    order2 = order.reshape(NW * GPW, 16)[_GIDX_FLAT].reshape(-1)

    combo = jnp.concatenate(
        [bbox_mu, bbox_log_sigma, pos_points, gt_centers,
         pos_strides[:, None], jnp.zeros((N, 3), jnp.float32)], axis=1)
    c_t = combo[order2].T

    rows_arr = (c_t[10:12, :].reshape(2, NW, GPW, 16)
                .transpose(1, 2, 0, 3))

    k5 = _knn_smallest5(keys2xm, rows_arr, sxb)
    k5_t = k5.transpose(2, 0, 1, 3).reshape(K, N)

    lc, lkl, lvar = _losses(c_t, k5_t, sigma_s.reshape(1))

    l_center = lc[0]
    l_kl = lkl[0]
    l_var = lvar[0]
    loss_total = (lambda_center * l_center + eff_lambda_kl * l_kl
                  + lambda_var * l_var)
    return (l_center, l_kl, l_var, loss_total)


# speedup vs baseline: 1.1831x; 1.1831x over previous
import numpy as np

import jax
import jax.numpy as jnp
from jax import lax
from jax.experimental import pallas as pl
from jax.experimental.pallas import tpu as pltpu
from jax.experimental.pallas import tpu_sc as plsc

N = 16384
M = 5000
M_PAD = 5120
NCH = M_PAD // 16
KST = 32
CPS = NCH // KST
K = 5
NC = 2
NS = 16
NW = NC * NS
RPW = N // NW
GPW = RPW // 16
R_OUT = 6
R_IN = 3
MASK_SENTINEL = 1e8

Q_STRIPE = 256
NSTR = N // Q_STRIPE
LV = Q_STRIPE // 16
_GIDX = np.empty((NW, GPW), np.int32)
for _s in range(NSTR):
    for _j in range(LV):
        _GIDX[(_s + (NW // LV) * _j) % NW,
              (_s // NW) * LV + _j] = _s * LV + _j
_GIDX_FLAT = _GIDX.reshape(-1)

_GDN = lax.GatherDimensionNumbers(
    offset_dims=(), collapsed_slice_dims=(0,), start_index_map=(0,))


def _shuffle(v, idx):
    return lax.gather(v, idx[:, None], _GDN, slice_sizes=(1,),
                      mode=lax.GatherScatterMode.PROMISE_IN_BOUNDS)


def _knn_body(keys_hbm, rows_hbm, sxb_hbm, out_hbm, keys_v, rows_v, sxb_v,
              out_v, msc_v):
    wid = lax.axis_index("c") * NS + lax.axis_index("s")
    pltpu.sync_copy(keys_hbm, keys_v)
    pltpu.sync_copy(rows_hbm.at[wid], rows_v)
    pltpu.sync_copy(sxb_hbm, sxb_v)

    ii = lax.iota(jnp.int32, 16)
    iz = ii * 0
    i15 = iz + 15
    one = jnp.int32(1)
    zero = jnp.int32(0)

    def any_lanes(fl, fr):
        v = fl * 32 + fr
        for s in (8, 4, 2, 1):
            v = v + _shuffle(v, jnp.bitwise_xor(ii, s))
        tot = v[0]
        return jnp.minimum(tot // 32, 1), jnp.minimum(tot % 32, 1)

    def sxb(row2, s):
        v = sxb_v[row2 + s // 16, :]
        return _shuffle(v, iz + s % 16)

    def group_body(slot, _):
        qx = rows_v[slot, 0, :]
        qy = rows_v[slot, 1, :]
        qmx = qx[8]
        qmy = qy[8]

        def chunk_chain(c, _=0):
            kxv = keys_v[0, pl.ds(c * 16, 16)]
            kyv = keys_v[1, pl.ds(c * 16, 16)]
            m0 = msc_v[0, :]
            m1 = msc_v[1, :]
            m2 = msc_v[2, :]
            m3 = msc_v[3, :]
            m4 = msc_v[4, :]
            for i in range(16):
                dx = qx - kxv[i]
                dy = qy - kyv[i]
                d2 = dx * dx + dy * dy
                v = jnp.where(d2 < 1e-4, MASK_SENTINEL, d2)
                n0 = jnp.minimum(m0, v)
                v = jnp.maximum(m0, v)
                n1 = jnp.minimum(m1, v)
                v = jnp.maximum(m1, v)
                n2 = jnp.minimum(m2, v)
                v = jnp.maximum(m2, v)
                n3 = jnp.minimum(m3, v)
                v = jnp.maximum(m3, v)
                m4 = jnp.minimum(m4, v)
                m0, m1, m2, m3 = n0, n1, n2, n3
            msc_v[0, :] = m0
            msc_v[1, :] = m1
            msc_v[2, :] = m2
            msc_v[3, :] = m3
            msc_v[4, :] = m4
            return 0

        def visit_stripe(s, _=0):
            xg = jnp.maximum(jnp.maximum(sxb(0, s) - qx, qx - sxb(2, s)),
                             0.0)
            xg2 = xg * xg
            base = s * CPS
            lo = zero
            hi = jnp.int32(CPS - 1)
            for _u in range(4):
                mid = (lo + hi + one) // 2
                kch = keys_v[1, pl.ds((base + mid) * 16, 16)]
                fge = jnp.where(kch <= qmy, one, zero)[0]
                lo = lo + fge * (mid - lo)
                hi = mid - one + fge * (hi - mid + one)
            c0 = lo
            chunk_chain(base + c0)

            def in_round(r, st):
                cl, cr = st
                m4 = msc_v[4, :]
                cap = jnp.int32(2) << jnp.minimum(r, 2)
                crc = jnp.minimum(cr, jnp.int32(CPS - 1))
                ylo = _shuffle(keys_v[1, pl.ds((base + crc) * 16, 16)], iz)
                dyr = jnp.maximum(ylo - qy, 0.0)
                validr = jnp.where(cr <= CPS - 1, one, zero)
                fr = jnp.where(xg2 + dyr * dyr < m4, validr, zero)
                clc = jnp.maximum(cl, zero)
                yhi = _shuffle(keys_v[1, pl.ds((base + clc) * 16, 16)], i15)
                dyl = jnp.maximum(qy - yhi, 0.0)
                validl = jnp.where(cl >= 0, one, zero)
                fl = jnp.where(xg2 + dyl * dyl < m4, validl, zero)
                needl, needr = any_lanes(fl, fr)
                nr = jnp.minimum(needr * cap, jnp.int32(CPS) - cr)
                nl = jnp.minimum(needl * cap, cl + one)
                lax.fori_loop(0, nr,
                              lambda i, m: chunk_chain(base + cr + i), 0)
                lax.fori_loop(0, nl,
                              lambda i, m: chunk_chain(base + cl - i), 0)
                return (cl - nl, cr + nr)

            lax.fori_loop(0, R_IN, in_round, (c0 - one, c0 + one))
            return 0

        f = (jnp.where(sxb_v[0, :] <= qmx, one, zero)
             + jnp.where(sxb_v[1, :] <= qmx, one, zero))
        for sh in (8, 4, 2, 1):
            f = f + _shuffle(f, jnp.bitwise_xor(ii, sh))
        s0 = jnp.clip(f[0] - one, 0, KST - 1)

        inf = jnp.full((16,), jnp.inf, jnp.float32)
        for t in range(K):
            msc_v[t, :] = inf
        visit_stripe(s0)

        def out_round(r, st):
            sl, sr = st
            m4 = msc_v[4, :]
            cap = one << jnp.minimum(r, 5)
            src = jnp.minimum(sr, jnp.int32(KST - 1))
            dxr = jnp.maximum(sxb(0, src) - qx, 0.0)
            validr = jnp.where(sr <= KST - 1, one, zero)
            fr = jnp.where(dxr * dxr < m4, validr, zero)
            slc = jnp.maximum(sl, zero)
            dxl = jnp.maximum(qx - sxb(2, slc), 0.0)
            validl = jnp.where(sl >= 0, one, zero)
            fl = jnp.where(dxl * dxl < m4, validl, zero)
            needl, needr = any_lanes(fl, fr)
            nr = jnp.minimum(needr * cap, jnp.int32(KST) - sr)
            nl = jnp.minimum(needl * cap, sl + one)
            lax.fori_loop(0, nr, lambda i, m: visit_stripe(sr + i), 0)
            lax.fori_loop(0, nl, lambda i, m: visit_stripe(sl - i), 0)
            return (sl - nl, sr + nr)

        lax.fori_loop(0, R_OUT, out_round, (s0 - one, s0 + one))
        for t in range(K):
            out_v[slot, t, :] = msc_v[t, :]
        return 0

    lax.fori_loop(0, GPW, group_body, 0)
    pltpu.sync_copy(out_v, out_hbm.at[wid])


def _knn_smallest5(keys2xm, rows_arr, sxb):
    mesh = plsc.VectorSubcoreMesh(core_axis_name="c", subcore_axis_name="s",
                                  num_cores=NC, num_subcores=NS)
    fn = pl.kernel(
        _knn_body,
        out_type=jax.ShapeDtypeStruct((NW, GPW, K, 16), jnp.float32),
        mesh=mesh,
        scratch_types=[
            pltpu.VMEM((2, M_PAD), jnp.float32),
            pltpu.VMEM((GPW, 2, 16), jnp.float32),
            pltpu.VMEM((4, 16), jnp.float32),
            pltpu.VMEM((GPW, K, 16), jnp.float32),
            pltpu.VMEM((K, 16), jnp.float32),
        ],
    )
    return fn(keys2xm, rows_arr, sxb)


def _loss_body(c_t, k5_t, sig_s_ref, lc_ref, lkl_ref, lvar_ref):
    mu_t = c_t[0:4, :]
    ls_t = c_t[4:8, :]
    sigma_s = sig_s_ref[0]
    s = c_t[12:13, :]

    gtd = (c_t[10:12, :] - c_t[8:10, :]) / s
    d = mu_t[0:2, :] - gtd
    a = jnp.abs(d)
    sl1 = jnp.where(a < 1.0, 0.5 * d * d, a - 0.5)
    lc_ref[0] = jnp.sum(sl1) / (2.0 * N)

    k5 = k5_t[...]
    d5 = jnp.where(k5 > 5e7, MASK_SENTINEL, jnp.sqrt(k5))
    d_px = jnp.sum(d5, axis=0, keepdims=True) * (1.0 / K)
    d_norm = jnp.clip(d_px / s, 0.5, 16.0)
    log_d = jnp.log(d_norm)
    sigma_c = jnp.maximum(0.5 * d_norm, 1e-6)
    sig_s_eff = jnp.maximum(sigma_s, 1e-6)

    mu01 = mu_t[0:2, :]
    mu23 = mu_t[2:4, :]
    sq01 = jnp.maximum(jnp.exp(ls_t[0:2, :]), 1e-6)
    sq23 = jnp.maximum(jnp.exp(ls_t[2:4, :]), 1e-6)
    kl01 = (jnp.log(sigma_c / sq01)
            + (sq01 * sq01 + mu01 * mu01) / (2.0 * sigma_c * sigma_c) - 0.5)
    dm23 = mu23 - log_d
    kl23 = (jnp.log(sig_s_eff / sq23)
            + (sq23 * sq23 + dm23 * dm23) / (2.0 * sig_s_eff * sig_s_eff)
            - 0.5)
    kl_sample = jnp.minimum(jnp.sum(kl01, axis=0) + jnp.sum(kl23, axis=0),
                            50.0)
    lkl_ref[0] = jnp.sum(kl_sample) / N
    lvar_ref[0] = jnp.sum(jnp.exp(ls_t[0:2, :])) / (2.0 * N)


def _losses(c_t, k5_t, sigma_s):
    return pl.pallas_call(
        _loss_body,
        out_shape=[jax.ShapeDtypeStruct((1,), jnp.float32)] * 3,
        in_specs=[pl.BlockSpec(memory_space=pltpu.VMEM)] * 2
        + [pl.BlockSpec(memory_space=pltpu.SMEM)],
        out_specs=[pl.BlockSpec(memory_space=pltpu.SMEM)] * 3,
    )(c_t, k5_t, sigma_s)


def kernel(bbox_mu, bbox_log_sigma, pos_points, pos_strides, gt_centers,
           gt_centers_list, cur_iter):
    lambda_center = 1.0
    lambda_kl = 0.1
    lambda_kl_warmup = 0.02
    lambda_var = 0.01
    sigma_s_init = 1.0
    sigma_s_final = 0.4
    warmup_iters = 2000
    anneal_iters = 2000

    cur_iter_f = jnp.asarray(cur_iter, jnp.float32)
    ratio = jnp.clip((cur_iter_f - warmup_iters) / anneal_iters, 0.0, 1.0)
    eff_lambda_kl = lambda_kl_warmup + ratio * (lambda_kl - lambda_kl_warmup)
    sigma_s = sigma_s_init - ratio * (sigma_s_init - sigma_s_final)

    all_gt = jnp.reshape(gt_centers_list, (M, 2))
    keys_xs = jnp.pad(all_gt[jnp.argsort(all_gt[:, 0])],
                      ((0, M_PAD - M), (0, 0)),
                      constant_values=1e6)
    kstr = keys_xs.reshape(KST, CPS * 16, 2)
    sxb = jnp.stack([kstr[:, 0, 0].reshape(2, 16),
                     kstr[:, -1, 0].reshape(2, 16)]
                    ).reshape(4, 16)
    yo = jnp.argsort(kstr[:, :, 1], axis=1)
    keys2xm = (jnp.take_along_axis(kstr, yo[:, :, None], axis=1)
               .reshape(M_PAD, 2).T)

    o1 = jnp.argsort(gt_centers[:, 0]).reshape(NSTR, Q_STRIPE)
    o2 = jnp.argsort(gt_centers[:, 1][o1], axis=1)
    order = jnp.take_along_axis(o1, o2, axis=1).reshape(-1)
    order2 = order.reshape(NW * GPW, 16)[_GIDX_FLAT].reshape(-1)

    combo = jnp.concatenate(
        [bbox_mu, bbox_log_sigma, pos_points, gt_centers,
         pos_strides[:, None], jnp.zeros((N, 3), jnp.float32)], axis=1)
    c_t = combo[order2].T

    rows_arr = (c_t[10:12, :].reshape(2, NW, GPW, 16)
                .transpose(1, 2, 0, 3))

    k5 = _knn_smallest5(keys2xm, rows_arr, sxb)
    k5_t = k5.transpose(2, 0, 1, 3).reshape(K, N)

    lc, lkl, lvar = _losses(c_t, k5_t, sigma_s.reshape(1))

    l_center = lc[0]
    l_kl = lkl[0]
    l_var = lvar[0]
    loss_total = (lambda_center * l_center + eff_lambda_kl * l_kl
                  + lambda_var * l_var)
    return (l_center, l_kl, l_var, loss_total)
